# Initial kernel scaffold; baseline (speedup 1.0000x reference)
import jax
import jax.numpy as jnp
from jax.experimental import pallas as pl


def kernel(ref_feat, e_kernel, e_ref, e_query, num_queries, kernel):
    raise NotImplementedError("write your pallas kernel here")


# speedup vs baseline: 4.8935x; 4.8935x over previous
import functools

import jax
import jax.numpy as jnp
from jax import lax
from jax.experimental import pallas as pl
from jax.experimental.pallas import tpu as pltpu
from jax.experimental.pallas import tpu_sc as plsc

NC = 2
NS = 16
NW = NC * NS


def _transform_tc(ref_feat, w):
    K0, D1, D2 = w.shape
    N = ref_feat.shape[0]

    def body(x_ref, w_ref, o_ref):
        o_ref[0] = jnp.dot(x_ref[...], w_ref[0],
                           preferred_element_type=jnp.float32,
                           precision=lax.Precision.HIGHEST)

    return pl.pallas_call(
        body,
        grid=(K0,),
        in_specs=[
            pl.BlockSpec((N, D1), lambda k: (0, 0)),
            pl.BlockSpec((1, D1, D2), lambda k: (k, 0, 0)),
        ],
        out_specs=pl.BlockSpec((1, N, D2), lambda k: (k, 0, 0)),
        out_shape=jax.ShapeDtypeStruct((K0, N, D2), jnp.float32),
    )(ref_feat, w)


def _scatter_sc(tr2d, gidx, qidx, Npad, D2):
    _, NCH, C = gidx.shape
    rows_per_tile = Npad // NS
    ZR = 64
    ZFULL = rows_per_tile // ZR
    ZREM = rows_per_tile - ZFULL * ZR
    mesh = plsc.VectorSubcoreMesh(core_axis_name="c", subcore_axis_name="s")

    @functools.partial(
        pl.kernel,
        mesh=mesh,
        out_type=jax.ShapeDtypeStruct((NC, Npad, D2), jnp.float32),
        scratch_types=[
            pltpu.VMEM((NCH, C), jnp.int32),
            pltpu.VMEM((NCH, C), jnp.int32),
            pltpu.VMEM((C, D2), jnp.float32),
            pltpu.VMEM((ZR, D2), jnp.float32),
            pltpu.VMEM_SHARED((Npad, D2), jnp.float32),
            pltpu.SemaphoreType.DMA,
        ],
    )
    def body(tr_hbm, gidx_hbm, qidx_hbm, out_hbm,
             gidx_v, qidx_v, rows_v, zbuf, acc, sem):
        cid = lax.axis_index("c")
        sid = lax.axis_index("s")
        wid = sid * NC + cid
        base = sid * rows_per_tile

        zvec = jnp.zeros((16,), jnp.float32)

        def zfill(r, carry):
            for j in range(D2 // 16):
                zbuf[r, pl.ds(j * 16, 16)] = zvec
            return carry

        lax.fori_loop(0, ZR, zfill, 0)

        def zcopy(i, carry):
            pltpu.sync_copy(zbuf, acc.at[pl.ds(base + i * ZR, ZR)])
            return carry

        lax.fori_loop(0, ZFULL, zcopy, 0)
        if ZREM:
            pltpu.sync_copy(zbuf.at[pl.ds(0, ZREM)],
                            acc.at[pl.ds(base + ZFULL * ZR, ZREM)])
        plsc.subcore_barrier()

        pltpu.sync_copy(gidx_hbm.at[wid], gidx_v)
        pltpu.sync_copy(qidx_hbm.at[wid], qidx_v)

        def chunk(i, carry):
            pltpu.async_copy(tr_hbm.at[gidx_v.at[i]], rows_v, sem).wait()
            pltpu.sync_copy(rows_v, acc.at[qidx_v.at[i]], add=True)
            return carry

        lax.fori_loop(0, NCH, chunk, 0)
        plsc.subcore_barrier()

        pltpu.sync_copy(acc.at[pl.ds(base, rows_per_tile)],
                        out_hbm.at[cid, pl.ds(base, rows_per_tile)])

    return body(tr2d, gidx, qidx)


def _add_tc(parts, N):
    _, Npad, D2 = parts.shape

    def body(p_ref, o_ref):
        o_ref[...] = p_ref[0, :N] + p_ref[1, :N]

    return pl.pallas_call(
        body,
        out_shape=jax.ShapeDtypeStruct((N, D2), jnp.float32),
    )(parts)


def kernel(ref_feat, e_kernel, e_ref, e_query, num_queries, kernel):
    w = kernel
    N, D1 = ref_feat.shape
    K0, _, D2 = w.shape
    E = e_ref.shape[0]
    C = 128
    EW = E // NW
    EWP = ((EW + C - 1) // C) * C
    NCH = EWP // C
    Npad = ((N + NS * 8 - 1) // (NS * 8)) * NS * 8

    transformed = _transform_tc(ref_feat, w)
    tr2d = transformed.reshape(K0 * N, D2)

    g = (e_kernel.astype(jnp.int32) * N + e_ref.astype(jnp.int32))
    gidx = jnp.pad(g.reshape(NW, EW), ((0, 0), (0, EWP - EW)),
                   constant_values=0).reshape(NW, NCH, C)
    qidx = jnp.pad(e_query.astype(jnp.int32).reshape(NW, EW),
                   ((0, 0), (0, EWP - EW)),
                   constant_values=N).reshape(NW, NCH, C)

    parts = _scatter_sc(tr2d, gidx, qidx, Npad, D2)
    return _add_tc(parts, N)
